# baseline (speedup 1.0000x reference)
import functools

import jax
import jax.numpy as jnp
from jax import lax
from jax.experimental import pallas as pl
from jax.experimental.pallas import tpu as pltpu
from jax.experimental.pallas import tpu_sc as plsc

N = 10000
E = 320000
IN_DIM = 128
HID = 256
HALF = 64

NC = 2
NS = 16
NW = NC * NS
CHUNK = 128
CHUNKS_PER_TILE = 80
E_PER_TILE = CHUNKS_PER_TILE * CHUNK
E_PAD = NW * E_PER_TILE
NPAD = 10240
ROWS_PER_TILE = NPAD // NS


def _mlp_body(x_ref, w0_ref, b0_ref, w1_ref, b1_ref, o_ref):
    h = jnp.dot(x_ref[...], w0_ref[...], preferred_element_type=jnp.float32)
    h = jnp.maximum(h + b0_ref[...], 0.0)
    o_ref[...] = jnp.dot(h, w1_ref[...], preferred_element_type=jnp.float32) + b1_ref[...]


def _mlp(x, W0, b0, W1, b1):
    BLK = 1000
    return pl.pallas_call(
        _mlp_body,
        grid=(N // BLK,),
        in_specs=[
            pl.BlockSpec((BLK, IN_DIM), lambda i: (i, 0)),
            pl.BlockSpec((IN_DIM, HID), lambda i: (0, 0)),
            pl.BlockSpec((1, HID), lambda i: (0, 0)),
            pl.BlockSpec((HID, HALF), lambda i: (0, 0)),
            pl.BlockSpec((1, HALF), lambda i: (0, 0)),
        ],
        out_specs=pl.BlockSpec((BLK, HALF), lambda i: (i, 0)),
        out_shape=jax.ShapeDtypeStruct((N, HALF), jnp.float32),
    )(x, W0, b0.reshape(1, HID), W1, b1.reshape(1, HALF))


def _sc_scatter(h, src3, dst3, zeros):
    mesh = plsc.VectorSubcoreMesh(core_axis_name="c", subcore_axis_name="s")

    @functools.partial(
        pl.kernel,
        mesh=mesh,
        compiler_params=pltpu.CompilerParams(use_tc_tiling_on_sc=False),
        out_type=jax.ShapeDtypeStruct((NC, NPAD, HALF), jnp.float32),
        scratch_types=[
            pltpu.VMEM((CHUNKS_PER_TILE, CHUNK), jnp.int32),
            pltpu.VMEM((CHUNKS_PER_TILE, CHUNK), jnp.int32),
            pltpu.VMEM((CHUNK, HALF), jnp.float32),
            pltpu.VMEM((CHUNK, HALF), jnp.float32),
            pltpu.VMEM_SHARED((NPAD, HALF), jnp.float32),
            pltpu.SemaphoreType.DMA,
            pltpu.SemaphoreType.DMA,
        ],
    )
    def k(h_hbm, src_hbm, dst_hbm, z_hbm, out_hbm,
          src_v, dst_v, rows0, rows1, accum, sem0, sem1):
        cid = lax.axis_index("c")
        sid = lax.axis_index("s")
        wid = sid * NC + cid

        pltpu.sync_copy(z_hbm.at[pl.ds(sid * ROWS_PER_TILE, ROWS_PER_TILE)],
                        accum.at[pl.ds(sid * ROWS_PER_TILE, ROWS_PER_TILE)])
        plsc.subcore_barrier()

        pltpu.sync_copy(src_hbm.at[wid], src_v)
        pltpu.sync_copy(dst_hbm.at[wid], dst_v)

        half_n = CHUNKS_PER_TILE // 2
        pltpu.async_copy(h_hbm.at[src_v.at[0]], rows0, sem0)

        def chunk_body(t, carry):
            j0 = 2 * t
            j1 = j0 + 1
            pltpu.make_async_copy(h_hbm.at[src_v.at[j0]], rows0, sem0).wait()
            pltpu.async_copy(h_hbm.at[src_v.at[j1]], rows1, sem1)
            pltpu.sync_copy(rows0, accum.at[dst_v.at[j0]], add=True)

            pltpu.make_async_copy(h_hbm.at[src_v.at[j1]], rows1, sem1).wait()

            @pl.when(t < half_n - 1)
            def _():
                pltpu.async_copy(h_hbm.at[src_v.at[j0 + 2]], rows0, sem0)

            pltpu.sync_copy(rows1, accum.at[dst_v.at[j1]], add=True)
            return carry

        lax.fori_loop(0, half_n, chunk_body, 0)
        plsc.subcore_barrier()

        pltpu.sync_copy(accum.at[pl.ds(sid * ROWS_PER_TILE, ROWS_PER_TILE)],
                        out_hbm.at[cid, pl.ds(sid * ROWS_PER_TILE, ROWS_PER_TILE)])

    return k(h, src3, dst3, zeros)


def _concat_body(h_ref, p_ref, o_ref):
    o_ref[:, :HALF] = h_ref[...]
    o_ref[:, HALF:] = p_ref[0] + p_ref[1]


def _concat(h, partials):
    BLK = 1000
    return pl.pallas_call(
        _concat_body,
        grid=(N // BLK,),
        in_specs=[
            pl.BlockSpec((BLK, HALF), lambda i: (i, 0)),
            pl.BlockSpec((NC, BLK, HALF), lambda i: (0, i, 0)),
        ],
        out_specs=pl.BlockSpec((BLK, 2 * HALF), lambda i: (i, 0)),
        out_shape=jax.ShapeDtypeStruct((N, 2 * HALF), jnp.float32),
    )(h, partials)


def kernel(features, edge_index, W0, b0, W1, b1):
    h = _mlp(features, W0, b0, W1, b1)

    pad = E_PAD - E
    src = jnp.concatenate([edge_index[0], jnp.zeros((pad,), jnp.int32)])
    dst = jnp.concatenate([edge_index[1], jnp.full((pad,), N, jnp.int32)])
    src3 = src.reshape(NW, CHUNKS_PER_TILE, CHUNK)
    dst3 = dst.reshape(NW, CHUNKS_PER_TILE, CHUNK)
    zeros = jnp.zeros((NPAD, HALF), jnp.float32)

    partials = _sc_scatter(h, src3, dst3, zeros)
    return _concat(h, partials)


# speedup vs baseline: 1.5525x; 1.5525x over previous
import functools

import jax
import jax.numpy as jnp
from jax import lax
from jax.experimental import pallas as pl
from jax.experimental.pallas import tpu as pltpu
from jax.experimental.pallas import tpu_sc as plsc

N = 10000
E = 320000
IN_DIM = 128
HID = 256
HALF = 64

NC = 2
NS = 16
NW = NC * NS
CHUNK = 128
CHUNKS_PER_TILE = 80
E_PER_TILE = CHUNKS_PER_TILE * CHUNK
E_PAD = NW * E_PER_TILE
NPAD = 10240
ROWS_PER_TILE = NPAD // NS


def _mlp_body(x_ref, w0_ref, b0_ref, w1_ref, b1_ref, o_ref):
    h = jnp.dot(x_ref[...], w0_ref[...], preferred_element_type=jnp.float32)
    h = jnp.maximum(h + b0_ref[...], 0.0)
    o_ref[...] = jnp.dot(h, w1_ref[...], preferred_element_type=jnp.float32) + b1_ref[...]


def _mlp(x, W0, b0, W1, b1):
    BLK = 1024
    return pl.pallas_call(
        _mlp_body,
        grid=(NPAD // BLK,),
        in_specs=[
            pl.BlockSpec((BLK, IN_DIM), lambda i: (i, 0)),
            pl.BlockSpec((IN_DIM, HID), lambda i: (0, 0)),
            pl.BlockSpec((1, HID), lambda i: (0, 0)),
            pl.BlockSpec((HID, HALF), lambda i: (0, 0)),
            pl.BlockSpec((1, HALF), lambda i: (0, 0)),
        ],
        out_specs=pl.BlockSpec((BLK, HALF), lambda i: (i, 0)),
        out_shape=jax.ShapeDtypeStruct((NPAD, HALF), jnp.float32),
    )(x, W0, b0.reshape(1, HID), W1, b1.reshape(1, HALF))


def _sc_scatter(h, src3, dst3, zeros):
    mesh = plsc.VectorSubcoreMesh(core_axis_name="c", subcore_axis_name="s")

    @functools.partial(
        pl.kernel,
        mesh=mesh,
        compiler_params=pltpu.CompilerParams(use_tc_tiling_on_sc=False),
        out_type=jax.ShapeDtypeStruct((NC, NPAD, HALF), jnp.float32),
        scratch_types=[
            pltpu.VMEM((CHUNKS_PER_TILE, CHUNK), jnp.int32),
            pltpu.VMEM((CHUNKS_PER_TILE, CHUNK), jnp.int32),
            pltpu.VMEM((CHUNK, HALF), jnp.float32),
            pltpu.VMEM((CHUNK, HALF), jnp.float32),
            pltpu.VMEM_SHARED((NPAD, HALF), jnp.float32),
            pltpu.VMEM_SHARED((NPAD, HALF), jnp.float32),
            pltpu.SemaphoreType.DMA,
            pltpu.SemaphoreType.DMA,
        ],
    )
    def k(h_hbm, src_hbm, dst_hbm, z_hbm, out_hbm,
          src_v, dst_v, rows0, rows1, accum, h_sp, sem0, sem1):
        cid = lax.axis_index("c")
        sid = lax.axis_index("s")
        wid = sid * NC + cid

        sl = pl.ds(sid * ROWS_PER_TILE, ROWS_PER_TILE)
        pltpu.sync_copy(z_hbm.at[sl], accum.at[sl])
        pltpu.sync_copy(h_hbm.at[sl], h_sp.at[sl])
        plsc.subcore_barrier()

        pltpu.sync_copy(src_hbm.at[wid], src_v)
        pltpu.sync_copy(dst_hbm.at[wid], dst_v)

        def chunk_body(j, carry):
            pltpu.async_copy(h_sp.at[src_v.at[j]], rows0, sem0).wait()
            pltpu.sync_copy(rows0, accum.at[dst_v.at[j]], add=True)
            return carry

        lax.fori_loop(0, CHUNKS_PER_TILE, chunk_body, 0)
        plsc.subcore_barrier()

        pltpu.sync_copy(accum.at[pl.ds(sid * ROWS_PER_TILE, ROWS_PER_TILE)],
                        out_hbm.at[cid, pl.ds(sid * ROWS_PER_TILE, ROWS_PER_TILE)])

    return k(h, src3, dst3, zeros)


def _concat_body(h_ref, p_ref, o_ref):
    o_ref[:, :HALF] = h_ref[...]
    o_ref[:, HALF:] = p_ref[0] + p_ref[1]


def _concat(h, partials):
    BLK = 1000
    return pl.pallas_call(
        _concat_body,
        grid=(N // BLK,),
        in_specs=[
            pl.BlockSpec((BLK, HALF), lambda i: (i, 0)),
            pl.BlockSpec((NC, BLK, HALF), lambda i: (0, i, 0)),
        ],
        out_specs=pl.BlockSpec((BLK, 2 * HALF), lambda i: (i, 0)),
        out_shape=jax.ShapeDtypeStruct((N, 2 * HALF), jnp.float32),
    )(h, partials)


def kernel(features, edge_index, W0, b0, W1, b1):
    xpad = jnp.zeros((NPAD, IN_DIM), jnp.float32).at[:N].set(features)
    h = _mlp(xpad, W0, b0, W1, b1)

    pad = E_PAD - E
    src = jnp.concatenate([edge_index[0], jnp.zeros((pad,), jnp.int32)])
    dst = jnp.concatenate([edge_index[1], jnp.full((pad,), N, jnp.int32)])
    src3 = src.reshape(NW, CHUNKS_PER_TILE, CHUNK)
    dst3 = dst.reshape(NW, CHUNKS_PER_TILE, CHUNK)
    zeros = jnp.zeros((NPAD, HALF), jnp.float32)

    partials = _sc_scatter(h, src3, dst3, zeros)
    return _concat(h, partials)


# speedup vs baseline: 1.8870x; 1.2155x over previous
import functools

import jax
import jax.numpy as jnp
from jax import lax
from jax.experimental import pallas as pl
from jax.experimental.pallas import tpu as pltpu
from jax.experimental.pallas import tpu_sc as plsc

N = 10000
E = 320000
IN_DIM = 128
HID = 256
HALF = 64

NC = 2
NS = 16
NW = NC * NS
CHUNK = 128
CHUNKS_PER_TILE = 80
E_PER_TILE = CHUNKS_PER_TILE * CHUNK
E_PAD = NW * E_PER_TILE
NPAD = 10240
ROWS_PER_TILE = NPAD // NS


def _mlp_body(x_ref, w0_ref, b0_ref, w1_ref, b1_ref, o_ref):
    h = jnp.dot(x_ref[...], w0_ref[...], preferred_element_type=jnp.float32)
    h = jnp.maximum(h + b0_ref[...], 0.0)
    o_ref[...] = jnp.dot(h, w1_ref[...], preferred_element_type=jnp.float32) + b1_ref[...]


def _mlp(x, W0, b0, W1, b1):
    BLK = 1024
    return pl.pallas_call(
        _mlp_body,
        grid=(NPAD // BLK,),
        in_specs=[
            pl.BlockSpec((BLK, IN_DIM), lambda i: (i, 0)),
            pl.BlockSpec((IN_DIM, HID), lambda i: (0, 0)),
            pl.BlockSpec((1, HID), lambda i: (0, 0)),
            pl.BlockSpec((HID, HALF), lambda i: (0, 0)),
            pl.BlockSpec((1, HALF), lambda i: (0, 0)),
        ],
        out_specs=pl.BlockSpec((BLK, HALF), lambda i: (i, 0)),
        out_shape=jax.ShapeDtypeStruct((NPAD, HALF), jnp.float32),
    )(x, W0, b0.reshape(1, HID), W1, b1.reshape(1, HALF))


def _sc_scatter(h, src3, dst3, zeros):
    mesh = plsc.VectorSubcoreMesh(core_axis_name="c", subcore_axis_name="s")

    @functools.partial(
        pl.kernel,
        mesh=mesh,
        compiler_params=pltpu.CompilerParams(use_tc_tiling_on_sc=False),
        out_type=jax.ShapeDtypeStruct((NC, NPAD, HALF), jnp.float32),
        scratch_types=[
            pltpu.VMEM((CHUNKS_PER_TILE, CHUNK), jnp.int32),
            pltpu.VMEM((CHUNKS_PER_TILE, CHUNK), jnp.int32),
            pltpu.VMEM((CHUNK, HALF), jnp.float32),
            pltpu.VMEM((CHUNK, HALF), jnp.float32),
            pltpu.VMEM_SHARED((NPAD, HALF), jnp.float32),
            pltpu.VMEM_SHARED((NPAD, HALF), jnp.float32),
            pltpu.SemaphoreType.DMA,
            pltpu.SemaphoreType.DMA,
        ],
    )
    def k(h_hbm, src_hbm, dst_hbm, z_hbm, out_hbm,
          src_v, dst_v, rows0, rows1, accum, h_sp, sem0, sem1):
        cid = lax.axis_index("c")
        sid = lax.axis_index("s")
        wid = sid * NC + cid

        sl = pl.ds(sid * ROWS_PER_TILE, ROWS_PER_TILE)
        pltpu.sync_copy(z_hbm.at[sl], accum.at[sl])
        pltpu.sync_copy(h_hbm.at[sl], h_sp.at[sl])
        plsc.subcore_barrier()

        pltpu.sync_copy(src_hbm.at[wid], src_v)
        pltpu.sync_copy(dst_hbm.at[wid], dst_v)

        half_n = CHUNKS_PER_TILE // 2
        pltpu.async_copy(h_sp.at[src_v.at[0]], rows0, sem0)

        def chunk_body(t, carry):
            j0 = 2 * t
            j1 = j0 + 1
            pltpu.make_async_copy(h_sp.at[src_v.at[j0]], rows0, sem0).wait()
            pltpu.async_copy(h_sp.at[src_v.at[j1]], rows1, sem1)
            pltpu.sync_copy(rows0, accum.at[dst_v.at[j0]], add=True)

            pltpu.make_async_copy(h_sp.at[src_v.at[j1]], rows1, sem1).wait()

            @pl.when(t < half_n - 1)
            def _():
                pltpu.async_copy(h_sp.at[src_v.at[j0 + 2]], rows0, sem0)

            pltpu.sync_copy(rows1, accum.at[dst_v.at[j1]], add=True)
            return carry

        lax.fori_loop(0, half_n, chunk_body, 0)
        plsc.subcore_barrier()

        pltpu.sync_copy(accum.at[pl.ds(sid * ROWS_PER_TILE, ROWS_PER_TILE)],
                        out_hbm.at[cid, pl.ds(sid * ROWS_PER_TILE, ROWS_PER_TILE)])

    return k(h, src3, dst3, zeros)


def _concat_body(h_ref, p_ref, o_ref):
    o_ref[:, :HALF] = h_ref[...]
    o_ref[:, HALF:] = p_ref[0] + p_ref[1]


def _concat(h, partials):
    BLK = 1000
    return pl.pallas_call(
        _concat_body,
        grid=(N // BLK,),
        in_specs=[
            pl.BlockSpec((BLK, HALF), lambda i: (i, 0)),
            pl.BlockSpec((NC, BLK, HALF), lambda i: (0, i, 0)),
        ],
        out_specs=pl.BlockSpec((BLK, 2 * HALF), lambda i: (i, 0)),
        out_shape=jax.ShapeDtypeStruct((N, 2 * HALF), jnp.float32),
    )(h, partials)


def kernel(features, edge_index, W0, b0, W1, b1):
    xpad = jnp.zeros((NPAD, IN_DIM), jnp.float32).at[:N].set(features)
    h = _mlp(xpad, W0, b0, W1, b1)

    pad = E_PAD - E
    src = jnp.concatenate([edge_index[0], jnp.zeros((pad,), jnp.int32)])
    dst = jnp.concatenate([edge_index[1], jnp.full((pad,), N, jnp.int32)])
    src3 = src.reshape(NW, CHUNKS_PER_TILE, CHUNK)
    dst3 = dst.reshape(NW, CHUNKS_PER_TILE, CHUNK)
    zeros = jnp.zeros((NPAD, HALF), jnp.float32)

    partials = _sc_scatter(h, src3, dst3, zeros)
    return _concat(h, partials)


# speedup vs baseline: 2.3514x; 1.2461x over previous
import functools

import jax
import jax.numpy as jnp
from jax import lax
from jax.experimental import pallas as pl
from jax.experimental.pallas import tpu as pltpu
from jax.experimental.pallas import tpu_sc as plsc

N = 10000
E = 320000
IN_DIM = 128
HID = 256
HALF = 64

NC = 2
NS = 16
NW = NC * NS
CHUNK = 128
CHUNKS_PER_TILE = 80
E_PER_TILE = CHUNKS_PER_TILE * CHUNK
E_PAD = NW * E_PER_TILE
NPAD = 10240
ROWS_PER_TILE = NPAD // NS


def _mlp_body(x_ref, w0_ref, b0_ref, w1_ref, b1_ref, o_ref, obf_ref):
    h = jnp.dot(x_ref[...], w0_ref[...], preferred_element_type=jnp.float32)
    h = jnp.maximum(h + b0_ref[...], 0.0)
    r = jnp.dot(h, w1_ref[...], preferred_element_type=jnp.float32) + b1_ref[...]
    o_ref[...] = r
    obf_ref[...] = r.astype(jnp.bfloat16)


def _mlp(x, W0, b0, W1, b1):
    BLK = 1024
    return pl.pallas_call(
        _mlp_body,
        grid=(NPAD // BLK,),
        in_specs=[
            pl.BlockSpec((BLK, IN_DIM), lambda i: (i, 0)),
            pl.BlockSpec((IN_DIM, HID), lambda i: (0, 0)),
            pl.BlockSpec((1, HID), lambda i: (0, 0)),
            pl.BlockSpec((HID, HALF), lambda i: (0, 0)),
            pl.BlockSpec((1, HALF), lambda i: (0, 0)),
        ],
        out_specs=[pl.BlockSpec((BLK, HALF), lambda i: (i, 0)),
                   pl.BlockSpec((BLK, HALF), lambda i: (i, 0))],
        out_shape=[jax.ShapeDtypeStruct((NPAD, HALF), jnp.float32),
                   jax.ShapeDtypeStruct((NPAD, HALF), jnp.bfloat16)],
    )(x, W0, b0.reshape(1, HID), W1, b1.reshape(1, HALF))


def _sc_scatter(h, src3, dst3, zeros):
    mesh = plsc.VectorSubcoreMesh(core_axis_name="c", subcore_axis_name="s")

    @functools.partial(
        pl.kernel,
        mesh=mesh,
        compiler_params=pltpu.CompilerParams(use_tc_tiling_on_sc=False),
        out_type=jax.ShapeDtypeStruct((NC, NPAD, HALF), jnp.bfloat16),
        scratch_types=[
            pltpu.VMEM((CHUNKS_PER_TILE, CHUNK), jnp.int32),
            pltpu.VMEM((CHUNKS_PER_TILE, CHUNK), jnp.int32),
            pltpu.VMEM((CHUNK, HALF), jnp.bfloat16),
            pltpu.VMEM((CHUNK, HALF), jnp.bfloat16),
            pltpu.VMEM_SHARED((NPAD, HALF), jnp.bfloat16),
            pltpu.VMEM_SHARED((NPAD, HALF), jnp.bfloat16),
            pltpu.SemaphoreType.DMA,
            pltpu.SemaphoreType.DMA,
        ],
    )
    def k(h_hbm, src_hbm, dst_hbm, z_hbm, out_hbm,
          src_v, dst_v, rows0, rows1, accum, h_sp, sem0, sem1):
        cid = lax.axis_index("c")
        sid = lax.axis_index("s")
        wid = sid * NC + cid

        sl = pl.ds(sid * ROWS_PER_TILE, ROWS_PER_TILE)
        pltpu.sync_copy(z_hbm.at[sl], accum.at[sl])
        pltpu.sync_copy(h_hbm.at[sl], h_sp.at[sl])
        plsc.subcore_barrier()

        pltpu.sync_copy(src_hbm.at[wid], src_v)
        pltpu.sync_copy(dst_hbm.at[wid], dst_v)

        half_n = CHUNKS_PER_TILE // 2
        pltpu.async_copy(h_sp.at[src_v.at[0]], rows0, sem0)

        def chunk_body(t, carry):
            j0 = 2 * t
            j1 = j0 + 1
            pltpu.make_async_copy(h_sp.at[src_v.at[j0]], rows0, sem0).wait()
            pltpu.async_copy(h_sp.at[src_v.at[j1]], rows1, sem1)
            pltpu.sync_copy(rows0, accum.at[dst_v.at[j0]], add=True)

            pltpu.make_async_copy(h_sp.at[src_v.at[j1]], rows1, sem1).wait()

            @pl.when(t < half_n - 1)
            def _():
                pltpu.async_copy(h_sp.at[src_v.at[j0 + 2]], rows0, sem0)

            pltpu.sync_copy(rows1, accum.at[dst_v.at[j1]], add=True)
            return carry

        lax.fori_loop(0, half_n, chunk_body, 0)
        plsc.subcore_barrier()

        pltpu.sync_copy(accum.at[pl.ds(sid * ROWS_PER_TILE, ROWS_PER_TILE)],
                        out_hbm.at[cid, pl.ds(sid * ROWS_PER_TILE, ROWS_PER_TILE)])

    return k(h, src3, dst3, zeros)


def _concat_body(h_ref, p_ref, o_ref):
    o_ref[:, :HALF] = h_ref[...]
    o_ref[:, HALF:] = p_ref[0].astype(jnp.float32) + p_ref[1].astype(jnp.float32)


def _concat(h, partials):
    BLK = 1000
    return pl.pallas_call(
        _concat_body,
        grid=(N // BLK,),
        in_specs=[
            pl.BlockSpec((BLK, HALF), lambda i: (i, 0)),
            pl.BlockSpec((NC, BLK, HALF), lambda i: (0, i, 0)),
        ],
        out_specs=pl.BlockSpec((BLK, 2 * HALF), lambda i: (i, 0)),
        out_shape=jax.ShapeDtypeStruct((N, 2 * HALF), jnp.float32),
    )(h, partials)


def kernel(features, edge_index, W0, b0, W1, b1):
    xpad = jnp.zeros((NPAD, IN_DIM), jnp.float32).at[:N].set(features)
    h, hbf = _mlp(xpad, W0, b0, W1, b1)

    pad = E_PAD - E
    src = jnp.concatenate([edge_index[0], jnp.zeros((pad,), jnp.int32)])
    dst = jnp.concatenate([edge_index[1], jnp.full((pad,), N, jnp.int32)])
    src3 = src.reshape(NW, CHUNKS_PER_TILE, CHUNK)
    dst3 = dst.reshape(NW, CHUNKS_PER_TILE, CHUNK)
    zeros = jnp.zeros((NPAD, HALF), jnp.bfloat16)

    partials = _sc_scatter(hbf, src3, dst3, zeros)
    return _concat(h, partials)


# speedup vs baseline: 2.4523x; 1.0429x over previous
import functools

import jax
import jax.numpy as jnp
from jax import lax
from jax.experimental import pallas as pl
from jax.experimental.pallas import tpu as pltpu
from jax.experimental.pallas import tpu_sc as plsc

N = 10000
E = 320000
IN_DIM = 128
HID = 256
HALF = 64

NC = 2
NS = 16
NW = NC * NS
CHUNK = 128
CHUNKS_PER_TILE = 80
E_PER_TILE = CHUNKS_PER_TILE * CHUNK
E_PAD = NW * E_PER_TILE
NPAD = 10240
ROWS_PER_TILE = NPAD // NS


def _mlp_body(x_ref, w0_ref, b0_ref, w1_ref, b1_ref, o_ref, obf_ref):
    h = jnp.dot(x_ref[...], w0_ref[...], preferred_element_type=jnp.float32)
    h = jnp.maximum(h + b0_ref[...], 0.0)
    r = jnp.dot(h, w1_ref[...], preferred_element_type=jnp.float32) + b1_ref[...]
    o_ref[...] = r
    obf_ref[...] = r.astype(jnp.bfloat16)


def _mlp(x, W0, b0, W1, b1):
    BLK = 1024
    return pl.pallas_call(
        _mlp_body,
        grid=(NPAD // BLK,),
        in_specs=[
            pl.BlockSpec((BLK, IN_DIM), lambda i: (i, 0)),
            pl.BlockSpec((IN_DIM, HID), lambda i: (0, 0)),
            pl.BlockSpec((1, HID), lambda i: (0, 0)),
            pl.BlockSpec((HID, HALF), lambda i: (0, 0)),
            pl.BlockSpec((1, HALF), lambda i: (0, 0)),
        ],
        out_specs=[pl.BlockSpec((BLK, HALF), lambda i: (i, 0)),
                   pl.BlockSpec((BLK, HALF), lambda i: (i, 0))],
        out_shape=[jax.ShapeDtypeStruct((NPAD, HALF), jnp.float32),
                   jax.ShapeDtypeStruct((NPAD, HALF), jnp.bfloat16)],
    )(x, W0, b0.reshape(1, HID), W1, b1.reshape(1, HALF))


def _sc_scatter(h, src3, dst3, zeros):
    mesh = plsc.VectorSubcoreMesh(core_axis_name="c", subcore_axis_name="s")

    @functools.partial(
        pl.kernel,
        mesh=mesh,
        compiler_params=pltpu.CompilerParams(use_tc_tiling_on_sc=False),
        out_type=jax.ShapeDtypeStruct((NC, NPAD, HALF), jnp.bfloat16),
        scratch_types=[
            pltpu.VMEM((CHUNKS_PER_TILE, CHUNK), jnp.int32),
            pltpu.VMEM((CHUNKS_PER_TILE, CHUNK), jnp.int32),
            pltpu.VMEM((CHUNK, HALF), jnp.bfloat16),
            pltpu.VMEM((CHUNK, HALF), jnp.bfloat16),
            pltpu.VMEM((CHUNK, HALF), jnp.bfloat16),
            pltpu.VMEM((CHUNK, HALF), jnp.bfloat16),
            pltpu.VMEM_SHARED((NPAD, HALF), jnp.bfloat16),
            pltpu.VMEM_SHARED((NPAD, HALF), jnp.bfloat16),
            pltpu.SemaphoreType.DMA,
            pltpu.SemaphoreType.DMA,
            pltpu.SemaphoreType.DMA,
            pltpu.SemaphoreType.DMA,
            pltpu.SemaphoreType.DMA,
            pltpu.SemaphoreType.DMA,
            pltpu.SemaphoreType.DMA,
            pltpu.SemaphoreType.DMA,
        ],
    )
    def k(h_hbm, src_hbm, dst_hbm, z_hbm, out_hbm,
          src_v, dst_v, rows0, rows1, rows2, rows3, accum, h_sp,
          gsem0, gsem1, gsem2, gsem3, ssem0, ssem1, ssem2, ssem3):
        cid = lax.axis_index("c")
        sid = lax.axis_index("s")
        wid = sid * NC + cid

        sl = pl.ds(sid * ROWS_PER_TILE, ROWS_PER_TILE)
        pltpu.sync_copy(z_hbm.at[sl], accum.at[sl])
        pltpu.sync_copy(h_hbm.at[sl], h_sp.at[sl])
        plsc.subcore_barrier()

        pltpu.sync_copy(src_hbm.at[wid], src_v)
        pltpu.sync_copy(dst_hbm.at[wid], dst_v)

        rows = (rows0, rows1, rows2, rows3)
        gsem = (gsem0, gsem1, gsem2, gsem3)
        ssem = (ssem0, ssem1, ssem2, ssem3)
        ngrp = CHUNKS_PER_TILE // 4

        pltpu.async_copy(h_sp.at[src_v.at[0]], rows[0], gsem[0])
        pltpu.async_copy(h_sp.at[src_v.at[1]], rows[1], gsem[1])

        def group_body(t, carry):
            j_base = 4 * t
            for b in range(4):
                j = j_base + b
                b2 = (b + 2) % 4
                pltpu.make_async_copy(h_sp.at[src_v.at[j]], rows[b], gsem[b]).wait()
                pltpu.async_copy(rows[b], accum.at[dst_v.at[j]], ssem[b], add=True)

                @pl.when(j + 2 < CHUNKS_PER_TILE)
                def _():
                    @pl.when(j >= 2)
                    def _():
                        pltpu.make_async_copy(
                            rows[b2], accum.at[dst_v.at[j]], ssem[b2]).wait()
                    pltpu.async_copy(h_sp.at[src_v.at[j + 2]], rows[b2], gsem[b2])
            return carry

        lax.fori_loop(0, ngrp, group_body, 0)

        for jj in range(CHUNKS_PER_TILE - 4, CHUNKS_PER_TILE):
            b = jj % 4
            pltpu.make_async_copy(rows[b], accum.at[dst_v.at[0]], ssem[b]).wait()
        plsc.subcore_barrier()

        pltpu.sync_copy(accum.at[pl.ds(sid * ROWS_PER_TILE, ROWS_PER_TILE)],
                        out_hbm.at[cid, pl.ds(sid * ROWS_PER_TILE, ROWS_PER_TILE)])

    return k(h, src3, dst3, zeros)


def _concat_body(h_ref, p_ref, o_ref):
    o_ref[:, :HALF] = h_ref[...]
    o_ref[:, HALF:] = p_ref[0].astype(jnp.float32) + p_ref[1].astype(jnp.float32)


def _concat(h, partials):
    BLK = 1000
    return pl.pallas_call(
        _concat_body,
        grid=(N // BLK,),
        in_specs=[
            pl.BlockSpec((BLK, HALF), lambda i: (i, 0)),
            pl.BlockSpec((NC, BLK, HALF), lambda i: (0, i, 0)),
        ],
        out_specs=pl.BlockSpec((BLK, 2 * HALF), lambda i: (i, 0)),
        out_shape=jax.ShapeDtypeStruct((N, 2 * HALF), jnp.float32),
    )(h, partials)


def kernel(features, edge_index, W0, b0, W1, b1):
    xpad = jnp.zeros((NPAD, IN_DIM), jnp.float32).at[:N].set(features)
    h, hbf = _mlp(xpad, W0, b0, W1, b1)

    pad = E_PAD - E
    src = jnp.concatenate([edge_index[0], jnp.zeros((pad,), jnp.int32)])
    dst = jnp.concatenate([edge_index[1], jnp.full((pad,), N, jnp.int32)])
    src3 = src.reshape(NW, CHUNKS_PER_TILE, CHUNK)
    dst3 = dst.reshape(NW, CHUNKS_PER_TILE, CHUNK)
    zeros = jnp.zeros((NPAD, HALF), jnp.bfloat16)

    partials = _sc_scatter(hbf, src3, dst3, zeros)
    return _concat(h, partials)


# speedup vs baseline: 2.7099x; 1.1050x over previous
import functools

import jax
import jax.numpy as jnp
from jax import lax
from jax.experimental import pallas as pl
from jax.experimental.pallas import tpu as pltpu
from jax.experimental.pallas import tpu_sc as plsc

N = 10000
E = 320000
IN_DIM = 128
HID = 256
HALF = 64

NC = 2
NS = 16
NW = NC * NS
CHUNK = 80
CHUNKS_PER_TILE = 125
E_PER_TILE = CHUNKS_PER_TILE * CHUNK
NPAD = 10240
ROWS_PER_TILE = NPAD // NS


def _mlp_body(x_ref, w0_ref, b0_ref, w1_ref, b1_ref, o_ref, obf_ref):
    h = jnp.dot(x_ref[...], w0_ref[...], preferred_element_type=jnp.float32)
    h = jnp.maximum(h + b0_ref[...], 0.0)
    r = jnp.dot(h, w1_ref[...], preferred_element_type=jnp.float32) + b1_ref[...]
    o_ref[...] = r
    obf_ref[...] = r.astype(jnp.bfloat16)


def _mlp(x, W0, b0, W1, b1):
    BLK = 2000
    return pl.pallas_call(
        _mlp_body,
        grid=(N // BLK,),
        in_specs=[
            pl.BlockSpec((BLK, IN_DIM), lambda i: (i, 0)),
            pl.BlockSpec((IN_DIM, HID), lambda i: (0, 0)),
            pl.BlockSpec((1, HID), lambda i: (0, 0)),
            pl.BlockSpec((HID, HALF), lambda i: (0, 0)),
            pl.BlockSpec((1, HALF), lambda i: (0, 0)),
        ],
        out_specs=[pl.BlockSpec((BLK, HALF), lambda i: (i, 0)),
                   pl.BlockSpec((BLK, HALF), lambda i: (i, 0))],
        out_shape=[jax.ShapeDtypeStruct((NPAD, HALF), jnp.float32),
                   jax.ShapeDtypeStruct((NPAD, HALF), jnp.bfloat16)],
    )(x, W0, b0.reshape(1, HID), W1, b1.reshape(1, HALF))


def _sc_scatter(h, src3, dst3, zeros):
    mesh = plsc.VectorSubcoreMesh(core_axis_name="c", subcore_axis_name="s")

    @functools.partial(
        pl.kernel,
        mesh=mesh,
        compiler_params=pltpu.CompilerParams(use_tc_tiling_on_sc=False),
        out_type=jax.ShapeDtypeStruct((NC, NPAD, HALF), jnp.bfloat16),
        scratch_types=[
            pltpu.VMEM((CHUNKS_PER_TILE, CHUNK), jnp.int32),
            pltpu.VMEM((CHUNKS_PER_TILE, CHUNK), jnp.int32),
            pltpu.VMEM((CHUNK, HALF), jnp.bfloat16),
            pltpu.VMEM((CHUNK, HALF), jnp.bfloat16),
            pltpu.VMEM((CHUNK, HALF), jnp.bfloat16),
            pltpu.VMEM((CHUNK, HALF), jnp.bfloat16),
            pltpu.VMEM_SHARED((NPAD, HALF), jnp.bfloat16),
            pltpu.VMEM_SHARED((NPAD, HALF), jnp.bfloat16),
            pltpu.SemaphoreType.DMA,
            pltpu.SemaphoreType.DMA,
            pltpu.SemaphoreType.DMA,
            pltpu.SemaphoreType.DMA,
            pltpu.SemaphoreType.DMA,
            pltpu.SemaphoreType.DMA,
            pltpu.SemaphoreType.DMA,
            pltpu.SemaphoreType.DMA,
        ],
    )
    def k(h_hbm, src_hbm, dst_hbm, z_hbm, out_hbm,
          src_v, dst_v, rows0, rows1, rows2, rows3, accum, h_sp,
          gsem0, gsem1, gsem2, gsem3, ssem0, ssem1, ssem2, ssem3):
        cid = lax.axis_index("c")
        sid = lax.axis_index("s")
        wid = sid * NC + cid

        sl = pl.ds(sid * ROWS_PER_TILE, ROWS_PER_TILE)
        pltpu.sync_copy(z_hbm.at[sl], accum.at[sl])
        pltpu.sync_copy(h_hbm.at[sl], h_sp.at[sl])
        plsc.subcore_barrier()

        pltpu.sync_copy(src_hbm.at[wid], src_v)
        pltpu.sync_copy(dst_hbm.at[wid], dst_v)

        rows = (rows0, rows1, rows2, rows3)
        gsem = (gsem0, gsem1, gsem2, gsem3)
        ssem = (ssem0, ssem1, ssem2, ssem3)
        ngrp = (CHUNKS_PER_TILE - 1) // 4

        pltpu.async_copy(h_sp.at[src_v.at[0]], rows[0], gsem[0])
        pltpu.async_copy(h_sp.at[src_v.at[1]], rows[1], gsem[1])

        def group_body(t, carry):
            j_base = 4 * t
            for b in range(4):
                j = j_base + b
                b2 = (b + 2) % 4
                pltpu.make_async_copy(h_sp.at[src_v.at[j]], rows[b], gsem[b]).wait()
                pltpu.async_copy(rows[b], accum.at[dst_v.at[j]], ssem[b], add=True)

                @pl.when(j + 2 < CHUNKS_PER_TILE)
                def _():
                    @pl.when(j >= 2)
                    def _():
                        pltpu.make_async_copy(
                            rows[b2], accum.at[dst_v.at[j]], ssem[b2]).wait()
                    pltpu.async_copy(h_sp.at[src_v.at[j + 2]], rows[b2], gsem[b2])
            return carry

        lax.fori_loop(0, ngrp, group_body, 0)

        jt = CHUNKS_PER_TILE - 1
        bt = jt % 4
        pltpu.make_async_copy(h_sp.at[src_v.at[jt]], rows[bt], gsem[bt]).wait()
        pltpu.async_copy(rows[bt], accum.at[dst_v.at[jt]], ssem[bt], add=True)
        for jj in range(CHUNKS_PER_TILE - 4, CHUNKS_PER_TILE):
            b = jj % 4
            pltpu.make_async_copy(rows[b], accum.at[dst_v.at[0]], ssem[b]).wait()
        plsc.subcore_barrier()

        pltpu.sync_copy(accum.at[pl.ds(sid * ROWS_PER_TILE, ROWS_PER_TILE)],
                        out_hbm.at[cid, pl.ds(sid * ROWS_PER_TILE, ROWS_PER_TILE)])

    return k(h, src3, dst3, zeros)


def _concat_body(h_ref, p_ref, o_ref):
    o_ref[:, :HALF] = h_ref[...]
    o_ref[:, HALF:] = p_ref[0].astype(jnp.float32) + p_ref[1].astype(jnp.float32)


def _concat(h, partials):
    BLK = 1000
    return pl.pallas_call(
        _concat_body,
        grid=(N // BLK,),
        in_specs=[
            pl.BlockSpec((BLK, HALF), lambda i: (i, 0)),
            pl.BlockSpec((NC, BLK, HALF), lambda i: (0, i, 0)),
        ],
        out_specs=pl.BlockSpec((BLK, 2 * HALF), lambda i: (i, 0)),
        out_shape=jax.ShapeDtypeStruct((N, 2 * HALF), jnp.float32),
    )(h, partials)


def kernel(features, edge_index, W0, b0, W1, b1):
    h, hbf = _mlp(features, W0, b0, W1, b1)

    src3 = edge_index[0].reshape(NW, CHUNKS_PER_TILE, CHUNK)
    dst3 = edge_index[1].reshape(NW, CHUNKS_PER_TILE, CHUNK)
    zeros = jnp.zeros((NPAD, HALF), jnp.bfloat16)

    partials = _sc_scatter(hbf, src3, dst3, zeros)
    return _concat(h, partials)


# speedup vs baseline: 3.4331x; 1.2669x over previous
import functools

import jax
import jax.numpy as jnp
from jax import lax
from jax.experimental import pallas as pl
from jax.experimental.pallas import tpu as pltpu
from jax.experimental.pallas import tpu_sc as plsc

N = 10000
E = 320000
IN_DIM = 128
HID = 256
HALF = 64

NC = 2
NS = 16
NW = NC * NS
CHUNK = 128
NCHUNKS = E // CHUNK
BASE_CH = NCHUNKS // NW
EXTRA_W = NCHUNKS - BASE_CH * NW
NPAD = 10240
ROWS_PER_TILE = NPAD // NS


def _mlp_body(x_ref, w0_ref, b0_ref, w1_ref, b1_ref, o_ref, obf_ref):
    h = jnp.dot(x_ref[...], w0_ref[...], preferred_element_type=jnp.float32)
    h = jnp.maximum(h + b0_ref[...], 0.0)
    r = jnp.dot(h, w1_ref[...], preferred_element_type=jnp.float32) + b1_ref[...]
    o_ref[...] = r
    obf_ref[...] = r.astype(jnp.bfloat16)


def _mlp(x, W0, b0, W1, b1):
    BLK = 2000
    return pl.pallas_call(
        _mlp_body,
        grid=(N // BLK,),
        in_specs=[
            pl.BlockSpec((BLK, IN_DIM), lambda i: (i, 0)),
            pl.BlockSpec((IN_DIM, HID), lambda i: (0, 0)),
            pl.BlockSpec((1, HID), lambda i: (0, 0)),
            pl.BlockSpec((HID, HALF), lambda i: (0, 0)),
            pl.BlockSpec((1, HALF), lambda i: (0, 0)),
        ],
        out_specs=[pl.BlockSpec((BLK, HALF), lambda i: (i, 0)),
                   pl.BlockSpec((BLK, HALF), lambda i: (i, 0))],
        out_shape=[jax.ShapeDtypeStruct((NPAD, HALF), jnp.float32),
                   jax.ShapeDtypeStruct((NPAD, HALF), jnp.bfloat16)],
    )(x, W0, b0.reshape(1, HID), W1, b1.reshape(1, HALF))


def _sc_scatter(h, edges3):
    mesh = plsc.VectorSubcoreMesh(core_axis_name="c", subcore_axis_name="s")

    @functools.partial(
        pl.kernel,
        mesh=mesh,
        compiler_params=pltpu.CompilerParams(use_tc_tiling_on_sc=False),
        out_type=jax.ShapeDtypeStruct((NC, NPAD, HALF), jnp.bfloat16),
        scratch_types=[
            pltpu.VMEM((BASE_CH + 1, 2, CHUNK), jnp.int32),
            pltpu.VMEM((CHUNK, HALF), jnp.bfloat16),
            pltpu.VMEM((CHUNK, HALF), jnp.bfloat16),
            pltpu.VMEM((CHUNK, HALF), jnp.bfloat16),
            pltpu.VMEM((CHUNK, HALF), jnp.bfloat16),
            pltpu.VMEM_SHARED((NPAD, HALF), jnp.bfloat16),
            pltpu.VMEM_SHARED((NPAD, HALF), jnp.bfloat16),
            pltpu.SemaphoreType.DMA,
            pltpu.SemaphoreType.DMA,
            pltpu.SemaphoreType.DMA,
            pltpu.SemaphoreType.DMA,
            pltpu.SemaphoreType.DMA,
            pltpu.SemaphoreType.DMA,
            pltpu.SemaphoreType.DMA,
            pltpu.SemaphoreType.DMA,
        ],
    )
    def k(h_hbm, e_hbm, out_hbm,
          idx_v, rows0, rows1, rows2, rows3, accum, h_sp,
          gsem0, gsem1, gsem2, gsem3, ssem0, ssem1, ssem2, ssem3):
        cid = lax.axis_index("c")
        sid = lax.axis_index("s")
        wid = sid * NC + cid

        zv = jnp.zeros((32,), jnp.bfloat16)

        def zrow(r, carry):
            rows0[r, 0:32] = zv
            rows0[r, 32:64] = zv
            return carry

        lax.fori_loop(0, CHUNK, zrow, 0)

        sl = pl.ds(sid * ROWS_PER_TILE, ROWS_PER_TILE)
        pltpu.async_copy(h_hbm.at[sl], h_sp.at[sl], gsem1)
        ch0 = wid * BASE_CH
        pltpu.async_copy(e_hbm.at[pl.ds(ch0, BASE_CH)],
                         idx_v.at[pl.ds(0, BASE_CH)], gsem2)
        for i in range(ROWS_PER_TILE // CHUNK):
            pltpu.async_copy(
                rows0, accum.at[pl.ds(sid * ROWS_PER_TILE + i * CHUNK, CHUNK)],
                gsem3)
        for i in range(ROWS_PER_TILE // CHUNK):
            pltpu.make_async_copy(
                rows0, accum.at[pl.ds(sid * ROWS_PER_TILE + i * CHUNK, CHUNK)],
                gsem3).wait()
        pltpu.make_async_copy(h_hbm.at[sl], h_sp.at[sl], gsem1).wait()
        pltpu.make_async_copy(e_hbm.at[pl.ds(ch0, BASE_CH)],
                              idx_v.at[pl.ds(0, BASE_CH)], gsem2).wait()
        plsc.subcore_barrier()

        rows = (rows0, rows1, rows2, rows3)
        gsem = (gsem0, gsem1, gsem2, gsem3)
        ssem = (ssem0, ssem1, ssem2, ssem3)
        ngrp = BASE_CH // 4
        tail0 = 4 * ngrp

        pltpu.async_copy(h_sp.at[idx_v.at[0, 0]], rows[0], gsem[0])
        pltpu.async_copy(h_sp.at[idx_v.at[1, 0]], rows[1], gsem[1])

        def group_body(t, carry):
            j_base = 4 * t
            for b in range(4):
                j = j_base + b
                b2 = (b + 2) % 4
                pltpu.make_async_copy(h_sp.at[idx_v.at[j, 0]], rows[b], gsem[b]).wait()
                pltpu.async_copy(rows[b], accum.at[idx_v.at[j, 1]], ssem[b], add=True)

                @pl.when(j >= 2)
                def _():
                    pltpu.make_async_copy(
                        rows[b2], accum.at[idx_v.at[j, 1]], ssem[b2]).wait()

                @pl.when(j + 2 < BASE_CH)
                def _():
                    pltpu.async_copy(h_sp.at[idx_v.at[j + 2, 0]], rows[b2], gsem[b2])
            return carry

        lax.fori_loop(0, ngrp, group_body, 0)

        for j in (tail0, tail0 + 1):
            b = j % 4
            pltpu.make_async_copy(h_sp.at[idx_v.at[j, 0]], rows[b], gsem[b]).wait()
            pltpu.async_copy(rows[b], accum.at[idx_v.at[j, 1]], ssem[b], add=True)
        for jj in range(BASE_CH - 4, BASE_CH):
            b = jj % 4
            pltpu.make_async_copy(rows[b], accum.at[idx_v.at[0, 1]], ssem[b]).wait()

        @pl.when(wid < EXTRA_W)
        def _():
            xc = NW * BASE_CH + wid
            pltpu.sync_copy(e_hbm.at[pl.ds(xc, 1)], idx_v.at[pl.ds(0, 1)])
            pltpu.async_copy(h_sp.at[idx_v.at[0, 0]], rows0, gsem0).wait()
            pltpu.sync_copy(rows0, accum.at[idx_v.at[0, 1]], add=True)

        plsc.subcore_barrier()

        pltpu.sync_copy(accum.at[pl.ds(sid * ROWS_PER_TILE, ROWS_PER_TILE)],
                        out_hbm.at[cid, pl.ds(sid * ROWS_PER_TILE, ROWS_PER_TILE)])

    return k(h, edges3)


def _concat_body(h_ref, p_ref, o_ref):
    o_ref[:, :HALF] = h_ref[...]
    o_ref[:, HALF:] = p_ref[0].astype(jnp.float32) + p_ref[1].astype(jnp.float32)


def _concat(h, partials):
    BLK = 2000
    return pl.pallas_call(
        _concat_body,
        grid=(N // BLK,),
        in_specs=[
            pl.BlockSpec((BLK, HALF), lambda i: (i, 0)),
            pl.BlockSpec((NC, BLK, HALF), lambda i: (0, i, 0)),
        ],
        out_specs=pl.BlockSpec((BLK, 2 * HALF), lambda i: (i, 0)),
        out_shape=jax.ShapeDtypeStruct((N, 2 * HALF), jnp.float32),
    )(h, partials)


def kernel(features, edge_index, W0, b0, W1, b1):
    h, hbf = _mlp(features, W0, b0, W1, b1)

    edges3 = edge_index.reshape(2, NCHUNKS, CHUNK).transpose(1, 0, 2)
    partials = _sc_scatter(hbf, edges3)
    return _concat(h, partials)
